# baseline (speedup 1.0000x reference)
import jax
import jax.numpy as jnp
from jax import lax
from jax.experimental import pallas as pl
from jax.experimental.pallas import tpu as pltpu
from jax.experimental.pallas import tpu_sc as plsc

B, NC = 2, 8
H, W = 512, 512
C, H1, W1 = 128, 256, 256
N = H1 * W1
TEMP = 10.0
LOSS_WEIGHT = 0.1
EPS = 1e-8

NUM_CORES, NUM_SUBCORES, LANES = 2, 16, 16
NUM_TILES = NUM_CORES * NUM_SUBCORES
ROWS_PER_TILE = H1 // NUM_SUBCORES
PIX_PER_TILE = ROWS_PER_TILE * W1
CHUNK = 256
NCHUNK = PIX_PER_TILE // CHUNK
PV_PER_CHUNK = CHUNK // LANES
NKEY = NC * NC
NACC = 4


def _sc_body(cls_hbm, lab_hbm, con_hbm, accs_hbm, cnts_hbm,
             clsbuf, labbuf, keybuf, acc0, acc1, acc2, acc3,
             cntv, conbuf, sem0, sem1):
    accs_l = (acc0, acc1, acc2, acc3)
    cid = lax.axis_index("c")
    sid = lax.axis_index("s")
    b = cid
    wid = cid * NUM_SUBCORES + sid

    zero = jnp.zeros((LANES,), jnp.float32)
    iota = lax.iota(jnp.int32, LANES)
    col_even = iota * 2

    @pl.loop(0, NKEY * C // (8 * LANES))
    def _zacc(r):
        for l8 in range(8):
            for a in accs_l:
                a[pl.ds((r * 8 + l8) * LANES, LANES)] = zero
    for l4 in range(NKEY // LANES):
        cntv[pl.ds(l4 * LANES, LANES)] = zero

    row0 = sid * ROWS_PER_TILE

    @pl.loop(0, ROWS_PER_TILE)
    def _row(r):
        in_row = (row0 + r) * 2
        pltpu.sync_copy(cls_hbm.at[b, :, in_row, :], clsbuf)
        pltpu.sync_copy(lab_hbm.at[b, 0, in_row, :], labbuf)
        for g in range(W1 // LANES):
            cidx = col_even + (32 * g)
            labv = plsc.load_gather(labbuf, [cidx])
            best = plsc.load_gather(clsbuf, [jnp.zeros_like(cidx), cidx])
            bi = jnp.zeros((LANES,), jnp.int32)
            for ch in range(1, NC):
                v = plsc.load_gather(clsbuf, [jnp.full_like(cidx, ch), cidx])
                m = v > best
                best = jnp.where(m, v, best)
                bi = jnp.where(m, jnp.int32(ch), bi)
            key = bi * NC + labv
            keybuf[pl.ds(r * W1 + g * LANES, LANES)] = key * C
            plsc.addupdate_scatter(cntv, [key], jnp.ones((LANES,), jnp.float32))

    base_pix = sid * PIX_PER_TILE

    def chunk_copy(j, par):
        sem = sem0 if par == 0 else sem1
        return pltpu.make_async_copy(
            con_hbm.at[b, :, pl.ds(base_pix + j * CHUNK, CHUNK)],
            conbuf.at[par], sem)

    chunk_copy(0, 0).start()
    chunk_copy(1, 1).start()

    @pl.loop(0, NCHUNK, step=2)
    def _outer(jj):
        for par in range(2):
            j = jj + par
            chunk_copy(j, par).wait()

            @pl.loop(0, PV_PER_CHUNK)
            def _pv(pv):
                kv = keybuf[pl.ds(j * CHUNK + pv * LANES, LANES)]
                for c in range(C):
                    v = conbuf[par, c, pl.ds(pv * LANES, LANES)]
                    plsc.addupdate_scatter(accs_l[c % NACC], [kv + c], v)

            nj = j + 2

            @pl.when(nj < NCHUNK)
            def _():
                chunk_copy(nj, par).start()

    for k, a in enumerate(accs_l):
        pltpu.sync_copy(a, accs_hbm.at[wid, k])
    pltpu.sync_copy(cntv, cnts_hbm.at[wid])


def _sc_call(cls_score, label_i, con_flat):
    fn = pl.kernel(
        _sc_body,
        out_type=[
            jax.ShapeDtypeStruct((NUM_TILES, NACC, NKEY * C), jnp.float32),
            jax.ShapeDtypeStruct((NUM_TILES, NKEY), jnp.float32),
        ],
        mesh=plsc.VectorSubcoreMesh(core_axis_name="c", subcore_axis_name="s"),
        compiler_params=pltpu.CompilerParams(needs_layout_passes=False),
        scratch_types=[
            pltpu.VMEM((NC, W), jnp.float32),
            pltpu.VMEM((W,), jnp.int32),
            pltpu.VMEM((PIX_PER_TILE,), jnp.int32),
            pltpu.VMEM((NKEY * C,), jnp.float32),
            pltpu.VMEM((NKEY * C,), jnp.float32),
            pltpu.VMEM((NKEY * C,), jnp.float32),
            pltpu.VMEM((NKEY * C,), jnp.float32),
            pltpu.VMEM((NKEY,), jnp.float32),
            pltpu.VMEM((2, C, CHUNK), jnp.float32),
            pltpu.SemaphoreType.DMA,
            pltpu.SemaphoreType.DMA,
        ],
    )
    return fn(cls_score, label_i, con_flat)


def _cls_sum_body(x_ref, o_ref):
    @pl.when(pl.program_id(0) == 0)
    def _():
        o_ref[0, 0] = jnp.float32(0.0)

    o_ref[0, 0] += jnp.sum(x_ref[...])


def _cls_sum_call(cls_score):
    return pl.pallas_call(
        _cls_sum_body,
        grid=(B * NC,),
        in_specs=[pl.BlockSpec((1, 1, H, W), lambda i: (i // NC, i % NC, 0, 0))],
        out_specs=pl.BlockSpec(memory_space=pltpu.SMEM),
        out_shape=jax.ShapeDtypeStruct((1, 1), jnp.float32),
    )(cls_score)


def _final_body(accs_ref, cnts_ref, clssum_ref, o_ref):
    a32 = accs_ref[...]
    ct32 = cnts_ref[...]
    A = jnp.sum(a32.reshape(B, NUM_SUBCORES * NACC, NKEY * C), axis=1)
    ctf = jnp.sum(ct32.reshape(B, NUM_SUBCORES, NKEY), axis=1)
    con_sum = jnp.sum(A)

    A4 = A.reshape(B, NC, NC, C)
    ct = ctf.reshape(B, NC, NC)
    jj = lax.broadcasted_iota(jnp.int32, (NC, NC), 0)
    kk = lax.broadcasted_iota(jnp.int32, (NC, NC), 1)
    eye = (jj == kk)
    eyef = eye.astype(jnp.float32)

    cnt_tt = jnp.sum(ct * eyef[None], axis=2)
    ttsum = jnp.sum(A4 * eyef[None, :, :, None], axis=2)
    tt_mean = ttsum / jnp.maximum(cnt_tt, 1.0)[:, :, None]
    cr = A4 / jnp.maximum(ct, 1.0)[..., None]
    pos = jnp.broadcast_to(tt_mean[:, None, :, :], cr.shape)
    neg = jnp.where(
        jnp.broadcast_to((cnt_tt > 0)[:, :, None, None], cr.shape),
        jnp.broadcast_to(tt_mean[:, :, None, :], cr.shape),
        cr,
    )

    def nrm(x):
        return x / (jnp.sqrt(jnp.sum(x * x, axis=-1, keepdims=True)) + EPS)

    cn, pn, ngn = nrm(cr), nrm(pos), nrm(neg)
    sp = jnp.sum(cn * pn, axis=-1) * TEMP
    sn = jnp.sum(cn * ngn, axis=-1) * TEMP
    mx = jnp.maximum(sp, sn)
    lse = mx + jnp.log(jnp.exp(sp - mx) + jnp.exp(sn - mx))
    per_region = lse - sp

    presentf = (jnp.sum(ct, axis=1) > 0).astype(jnp.float32)
    validf = ((ct > 0).astype(jnp.float32)
              * (cnt_tt > 0).astype(jnp.float32)[:, None, :]
              * presentf[:, :, None]
              * (1.0 - eyef)[None])
    nvalid = jnp.sum(validf)
    loss = LOSS_WEIGHT * jnp.sum(per_region * validf) / jnp.maximum(nvalid, 1.0)
    fallback = (-clssum_ref[0, 0] + con_sum) * 1e-16
    o_ref[0, 0] = jnp.where(nvalid > 0, loss, fallback)


def _final_call(accs, cnts, cls_sum):
    return pl.pallas_call(
        _final_body,
        in_specs=[
            pl.BlockSpec(memory_space=pltpu.VMEM),
            pl.BlockSpec(memory_space=pltpu.VMEM),
            pl.BlockSpec(memory_space=pltpu.SMEM),
        ],
        out_specs=pl.BlockSpec(memory_space=pltpu.SMEM),
        out_shape=jax.ShapeDtypeStruct((1, 1), jnp.float32),
    )(accs, cnts, cls_sum)


def kernel(cls_score, label, con_seg_logit):
    label_i = label.astype(jnp.int32)
    con_flat = con_seg_logit.reshape(B, C, N)
    accs, cnts = _sc_call(cls_score, label_i, con_flat)
    cls_sum = _cls_sum_call(cls_score)
    out = _final_call(accs, cnts, cls_sum)
    return out[0, 0]


# speedup vs baseline: 2.3665x; 2.3665x over previous
import jax
import jax.numpy as jnp
from jax import lax
from jax.experimental import pallas as pl
from jax.experimental.pallas import tpu as pltpu
from jax.experimental.pallas import tpu_sc as plsc

B, NC = 2, 8
H, W = 512, 512
C, H1, W1 = 128, 256, 256
N = H1 * W1
TEMP = 10.0
LOSS_WEIGHT = 0.1
EPS = 1e-8

NUM_CORES, NUM_SUBCORES, LANES = 2, 16, 16
NUM_TILES = NUM_CORES * NUM_SUBCORES
ROWS_PER_TILE = H1 // NUM_SUBCORES
PIX_PER_TILE = ROWS_PER_TILE * W1
NKEY = NC * NC

PCHUNK = 128
NPCHUNK = PIX_PER_TILE // PCHUNK
TRBLK = 2048


def _tr_body(x_ref, o_ref):
    o_ref[...] = x_ref[0].T


def _tr_call(con_flat):
    return pl.pallas_call(
        _tr_body,
        grid=(B, N // TRBLK),
        in_specs=[pl.BlockSpec((1, C, TRBLK), lambda b, p: (b, 0, p))],
        out_specs=pl.BlockSpec((TRBLK, C), lambda b, p: (b * (N // TRBLK) + p, 0)),
        out_shape=jax.ShapeDtypeStruct((B * N, C), jnp.float32),
    )(con_flat)


def _sc_body(cls_hbm, lab_hbm, cont_hbm, accs_hbm, cnts_hbm,
             clsbuf, labbuf, keybuf, acc, cntv, conbuf, spacc, sem0, sem1):
    cid = lax.axis_index("c")
    sid = lax.axis_index("s")
    b = cid
    wid = cid * NUM_SUBCORES + sid

    zero = jnp.zeros((LANES,), jnp.float32)
    iota = lax.iota(jnp.int32, LANES)
    col_even = iota * 2

    @pl.loop(0, NKEY)
    def _zacc(r):
        for l8 in range(C // LANES):
            acc[r, pl.ds(l8 * LANES, LANES)] = zero
    for l4 in range(NKEY // LANES):
        cntv[pl.ds(l4 * LANES, LANES)] = zero

    @pl.when(sid == 0)
    def _():
        pltpu.sync_copy(acc, spacc)

    row0 = sid * ROWS_PER_TILE

    @pl.loop(0, ROWS_PER_TILE)
    def _row(r):
        in_row = (row0 + r) * 2
        pltpu.sync_copy(cls_hbm.at[b, :, in_row, :], clsbuf)
        pltpu.sync_copy(lab_hbm.at[b, 0, in_row, :], labbuf)
        for g in range(W1 // LANES):
            cidx = col_even + (32 * g)
            labv = plsc.load_gather(labbuf, [cidx])
            best = plsc.load_gather(clsbuf, [jnp.zeros_like(cidx), cidx])
            bi = jnp.zeros((LANES,), jnp.int32)
            for ch in range(1, NC):
                v = plsc.load_gather(clsbuf, [jnp.full_like(cidx, ch), cidx])
                m = v > best
                best = jnp.where(m, v, best)
                bi = jnp.where(m, jnp.int32(ch), bi)
            key = bi * NC + labv
            keybuf[r * 2 + g // 8, pl.ds((g % 8) * LANES, LANES)] = key
            plsc.addupdate_scatter(cntv, [key], jnp.ones((LANES,), jnp.float32))

    base_row = b * N + sid * PIX_PER_TILE

    def chunk_copy(j, par):
        sem = sem0 if par == 0 else sem1
        return pltpu.make_async_copy(
            cont_hbm.at[pl.ds(base_row + j * PCHUNK, PCHUNK)],
            conbuf.at[par], sem)

    chunk_copy(0, 0).start()
    chunk_copy(1, 1).start()
    plsc.subcore_barrier()

    @pl.loop(0, NPCHUNK, step=2)
    def _outer(jj):
        for par in range(2):
            j = jj + par
            chunk_copy(j, par).wait()
            pltpu.sync_copy(conbuf.at[par], spacc.at[keybuf.at[j]], add=True)
            nj = j + 2

            @pl.when(nj < NPCHUNK)
            def _():
                chunk_copy(nj, par).start()

    plsc.subcore_barrier()

    @pl.when(sid == 0)
    def _():
        pltpu.sync_copy(spacc, accs_hbm.at[b])

    pltpu.sync_copy(cntv, cnts_hbm.at[wid])


def _sc_call(cls_score, label_i, con_t):
    fn = pl.kernel(
        _sc_body,
        out_type=[
            jax.ShapeDtypeStruct((B, NKEY, C), jnp.float32),
            jax.ShapeDtypeStruct((NUM_TILES, NKEY), jnp.float32),
        ],
        mesh=plsc.VectorSubcoreMesh(core_axis_name="c", subcore_axis_name="s"),
        compiler_params=pltpu.CompilerParams(needs_layout_passes=False),
        scratch_types=[
            pltpu.VMEM((NC, W), jnp.float32),
            pltpu.VMEM((W,), jnp.int32),
            pltpu.VMEM((NPCHUNK, PCHUNK), jnp.int32),
            pltpu.VMEM((NKEY, C), jnp.float32),
            pltpu.VMEM((NKEY,), jnp.float32),
            pltpu.VMEM((2, PCHUNK, C), jnp.float32),
            pltpu.VMEM_SHARED((NKEY, C), jnp.float32),
            pltpu.SemaphoreType.DMA,
            pltpu.SemaphoreType.DMA,
        ],
    )
    return fn(cls_score, label_i, con_t)


def _cls_sum_body(x_ref, o_ref):
    @pl.when(pl.program_id(0) == 0)
    def _():
        o_ref[0, 0] = jnp.float32(0.0)

    o_ref[0, 0] += jnp.sum(x_ref[...])


def _cls_sum_call(cls_score):
    return pl.pallas_call(
        _cls_sum_body,
        grid=(B * NC,),
        in_specs=[pl.BlockSpec((1, 1, H, W), lambda i: (i // NC, i % NC, 0, 0))],
        out_specs=pl.BlockSpec(memory_space=pltpu.SMEM),
        out_shape=jax.ShapeDtypeStruct((1, 1), jnp.float32),
    )(cls_score)


def _final_body(accs_ref, cnts_ref, clssum_ref, o_ref):
    A = accs_ref[...]
    ct32 = cnts_ref[...]
    ctf = jnp.sum(ct32.reshape(B, NUM_SUBCORES, NKEY), axis=1)
    con_sum = jnp.sum(A)

    A4 = A.reshape(B, NC, NC, C)
    ct = ctf.reshape(B, NC, NC)
    jj = lax.broadcasted_iota(jnp.int32, (NC, NC), 0)
    kk = lax.broadcasted_iota(jnp.int32, (NC, NC), 1)
    eye = (jj == kk)
    eyef = eye.astype(jnp.float32)

    cnt_tt = jnp.sum(ct * eyef[None], axis=2)
    ttsum = jnp.sum(A4 * eyef[None, :, :, None], axis=2)
    tt_mean = ttsum / jnp.maximum(cnt_tt, 1.0)[:, :, None]
    cr = A4 / jnp.maximum(ct, 1.0)[..., None]
    pos = jnp.broadcast_to(tt_mean[:, None, :, :], cr.shape)
    neg = jnp.where(
        jnp.broadcast_to((cnt_tt > 0)[:, :, None, None], cr.shape),
        jnp.broadcast_to(tt_mean[:, :, None, :], cr.shape),
        cr,
    )

    def nrm(x):
        return x / (jnp.sqrt(jnp.sum(x * x, axis=-1, keepdims=True)) + EPS)

    cn, pn, ngn = nrm(cr), nrm(pos), nrm(neg)
    sp = jnp.sum(cn * pn, axis=-1) * TEMP
    sn = jnp.sum(cn * ngn, axis=-1) * TEMP
    mx = jnp.maximum(sp, sn)
    lse = mx + jnp.log(jnp.exp(sp - mx) + jnp.exp(sn - mx))
    per_region = lse - sp

    presentf = (jnp.sum(ct, axis=1) > 0).astype(jnp.float32)
    validf = ((ct > 0).astype(jnp.float32)
              * (cnt_tt > 0).astype(jnp.float32)[:, None, :]
              * presentf[:, :, None]
              * (1.0 - eyef)[None])
    nvalid = jnp.sum(validf)
    loss = LOSS_WEIGHT * jnp.sum(per_region * validf) / jnp.maximum(nvalid, 1.0)
    fallback = (-clssum_ref[0, 0] + con_sum) * 1e-16
    o_ref[0, 0] = jnp.where(nvalid > 0, loss, fallback)


def _final_call(accs, cnts, cls_sum):
    return pl.pallas_call(
        _final_body,
        in_specs=[
            pl.BlockSpec(memory_space=pltpu.VMEM),
            pl.BlockSpec(memory_space=pltpu.VMEM),
            pl.BlockSpec(memory_space=pltpu.SMEM),
        ],
        out_specs=pl.BlockSpec(memory_space=pltpu.SMEM),
        out_shape=jax.ShapeDtypeStruct((1, 1), jnp.float32),
    )(accs, cnts, cls_sum)


def kernel(cls_score, label, con_seg_logit):
    label_i = label.astype(jnp.int32)
    con_flat = con_seg_logit.reshape(B, C, N)
    con_t = _tr_call(con_flat)
    accs, cnts = _sc_call(cls_score, label_i, con_t)
    cls_sum = _cls_sum_call(cls_score)
    out = _final_call(accs, cnts, cls_sum)
    return out[0, 0]
